# baseline (speedup 1.0000x reference)
import jax
import jax.numpy as jnp
from jax.experimental import pallas as pl
from jax.experimental.pallas import tpu as pltpu

N = 4096
F = 64
BD = 256
NBLK = N // BD
PH = NBLK // 2


def _gcn_kernel(a_lo_ref, a_hi_ref, x_ref, w1_ref, b1_ref, w2_ref, b2_ref,
                out_ref, a8_ref, dinv_ref, m1t_ref, m2t_ref):
    i = pl.program_id(0)

    @pl.when(i < PH)
    def _():
        col = jax.lax.broadcasted_iota(jnp.int32, (BD, N), 1)
        row = jax.lax.broadcasted_iota(jnp.int32, (BD, N), 0) + 2 * i * BD
        a = a_lo_ref[...]
        ah_lo = jnp.where(row == col, 1.0, (a != 0).astype(jnp.float32))
        a8_ref[pl.ds(2 * i * BD, BD), :] = ah_lo.astype(jnp.bfloat16)
        a = a_hi_ref[...]
        ah_hi = jnp.where(row + BD == col, 1.0, (a != 0).astype(jnp.float32))
        a8_ref[pl.ds((2 * i + 1) * BD, BD), :] = ah_hi.astype(jnp.bfloat16)
        deg = (jnp.sum(ah_lo, axis=0, keepdims=True)
               + jnp.sum(ah_hi, axis=0, keepdims=True))
        @pl.when(i == 0)
        def _():
            dinv_ref[...] = deg
        @pl.when(i > 0)
        def _():
            dinv_ref[...] += deg

    @pl.when(i == PH - 1)
    def _():
        deg = dinv_ref[...]
        dinv_ref[...] = jnp.where(deg > 0, jax.lax.rsqrt(deg), 0.0)
        xwt = jax.lax.dot_general(
            w1_ref[...], x_ref[...], (((0,), (1,)), ((), ())),
            preferred_element_type=jnp.float32)
        m1t_ref[...] = (dinv_ref[...] * xwt).astype(jnp.bfloat16)

    @pl.when(i == PH)
    def _():
        acc = jnp.dot(m1t_ref[...], a8_ref[...],
                      preferred_element_type=jnp.float32)
        dinv = dinv_ref[...]
        h = jnp.maximum(dinv * acc + b1_ref[...], 0.0)
        m2t = jax.lax.dot_general(
            w2_ref[...], h, (((0,), (0,)), ((), ())),
            preferred_element_type=jnp.float32)
        m2t_ref[...] = (dinv * m2t).astype(jnp.bfloat16)

    @pl.when(i == PH + 1)
    def _():
        acc = jnp.dot(m2t_ref[...], a8_ref[...],
                      preferred_element_type=jnp.float32)
        acc_n = jnp.transpose(dinv_ref[...] * acc)
        p = acc_n + b2_ref[...] + x_ref[...]
        mx = jnp.max(p, axis=0, keepdims=True)
        e = jnp.exp(p - mx)
        s = jnp.sum(e, axis=0, keepdims=True)
        out_ref[...] = e / s


@jax.jit
def _run(A, x, W1, b1r, W2, b2r):
    out = pl.pallas_call(
        _gcn_kernel,
        grid=(PH + 2,),
        in_specs=[
            pl.BlockSpec((BD, N),
                         lambda i: (jnp.minimum(2 * i, NBLK - 2), 0)),
            pl.BlockSpec((BD, N),
                         lambda i: (jnp.minimum(2 * i + 1, NBLK - 1), 0)),
            pl.BlockSpec((N, F), lambda i: (0, 0)),
            pl.BlockSpec((F, F), lambda i: (0, 0)),
            pl.BlockSpec((F, 1), lambda i: (0, 0)),
            pl.BlockSpec((F, F), lambda i: (0, 0)),
            pl.BlockSpec((1, F), lambda i: (0, 0)),
        ],
        out_specs=pl.BlockSpec((N, F), lambda i: (0, 0)),
        out_shape=jax.ShapeDtypeStruct((N, F), jnp.float32),
        scratch_shapes=[
            pltpu.VMEM((N, N), jnp.bfloat16),
            pltpu.VMEM((1, N), jnp.float32),
            pltpu.VMEM((F, N), jnp.bfloat16),
            pltpu.VMEM((F, N), jnp.bfloat16),
        ],
    )(A, A, x, W1, b1r, W2, b2r)
    return out


def kernel(A, x, W1, b1, W2, b2, sigmoid_param):
    out = _run(A, x, W1, b1.reshape(F, 1), W2, b2.reshape(1, F))
    return out.astype(jnp.float64)


# speedup vs baseline: 1.0456x; 1.0456x over previous
import jax
import jax.numpy as jnp
from jax.experimental import pallas as pl
from jax.experimental.pallas import tpu as pltpu

N = 4096
F = 64
BD = 128
NS = 4
NBLK = N // BD
PH = NBLK // NS


def _gcn_kernel(a0_ref, a1_ref, a2_ref, a3_ref, xt_ref, w1t_ref, b1_ref,
                w2t_ref, b2_ref, out_ref, a8_ref, dinv_ref, m1t_ref,
                m2t_ref):
    i = pl.program_id(0)

    @pl.when(i < PH)
    def _():
        col = jax.lax.broadcasted_iota(jnp.int32, (BD, N), 1)
        row0 = jax.lax.broadcasted_iota(jnp.int32, (BD, N), 0) + NS * i * BD
        deg = jnp.zeros((1, N), jnp.float32)
        for s, a_ref in enumerate((a0_ref, a1_ref, a2_ref, a3_ref)):
            a = a_ref[...]
            ah = jnp.where(row0 + s * BD == col, 1.0,
                           (a != 0).astype(jnp.float32))
            a8_ref[pl.ds((NS * i + s) * BD, BD), :] = ah.astype(jnp.bfloat16)
            deg = deg + jnp.sum(ah, axis=0, keepdims=True)
        @pl.when(i == 0)
        def _():
            dinv_ref[...] = deg
        @pl.when(i > 0)
        def _():
            dinv_ref[...] += deg

    @pl.when(i == PH - 1)
    def _():
        deg = dinv_ref[...]
        dinv_ref[...] = jnp.where(deg > 0, jax.lax.rsqrt(deg), 0.0)
        xwt = jnp.dot(w1t_ref[...], xt_ref[...],
                      preferred_element_type=jnp.float32)
        m1t_ref[...] = (dinv_ref[...] * xwt).astype(jnp.bfloat16)

    @pl.when(i == PH)
    def _():
        acc = jnp.dot(m1t_ref[...], a8_ref[...],
                      preferred_element_type=jnp.float32)
        dinv = dinv_ref[...]
        h = jnp.maximum(dinv * acc + b1_ref[...], 0.0)
        m2t_ref[...] = (dinv * jnp.dot(
            w2t_ref[...], h, preferred_element_type=jnp.float32)
        ).astype(jnp.bfloat16)

    @pl.when(i == PH + 1)
    def _():
        acc = jnp.dot(m2t_ref[...], a8_ref[...],
                      preferred_element_type=jnp.float32)
        p = dinv_ref[...] * acc + b2_ref[...] + xt_ref[...]
        mx = jnp.max(p, axis=1, keepdims=True)
        e = jnp.exp(p - mx)
        s = jnp.sum(e, axis=1, keepdims=True)
        out_ref[...] = e / s


@jax.jit
def _run(A, xt, W1t, b1c, W2t, b2c):
    out_t = pl.pallas_call(
        _gcn_kernel,
        grid=(PH + 2,),
        in_specs=[
            pl.BlockSpec((BD, N),
                         lambda i: (jnp.minimum(NS * i, NBLK - NS), 0)),
            pl.BlockSpec((BD, N),
                         lambda i: (jnp.minimum(NS * i + 1, NBLK - NS + 1), 0)),
            pl.BlockSpec((BD, N),
                         lambda i: (jnp.minimum(NS * i + 2, NBLK - NS + 2), 0)),
            pl.BlockSpec((BD, N),
                         lambda i: (jnp.minimum(NS * i + 3, NBLK - NS + 3), 0)),
            pl.BlockSpec((F, N), lambda i: (0, 0)),
            pl.BlockSpec((F, F), lambda i: (0, 0)),
            pl.BlockSpec((F, 1), lambda i: (0, 0)),
            pl.BlockSpec((F, F), lambda i: (0, 0)),
            pl.BlockSpec((F, 1), lambda i: (0, 0)),
        ],
        out_specs=pl.BlockSpec((F, N), lambda i: (0, 0)),
        out_shape=jax.ShapeDtypeStruct((F, N), jnp.float32),
        scratch_shapes=[
            pltpu.VMEM((N, N), jnp.bfloat16),
            pltpu.VMEM((1, N), jnp.float32),
            pltpu.VMEM((F, N), jnp.bfloat16),
            pltpu.VMEM((F, N), jnp.bfloat16),
        ],
    )(A, A, A, A, xt, W1t, b1c, W2t, b2c)
    return out_t


def kernel(A, x, W1, b1, W2, b2, sigmoid_param):
    out_t = _run(A, x.T, W1.T, b1.reshape(F, 1), W2.T, b2.reshape(F, 1))
    return out_t.T.astype(jnp.float64)


# speedup vs baseline: 1.1514x; 1.1012x over previous
import jax
import jax.numpy as jnp
from jax.experimental import pallas as pl
from jax.experimental.pallas import tpu as pltpu

N = 4096
F = 64
BD = 256
NBLK = N // BD
PH = NBLK // 2


def _gcn_kernel(a_lo_ref, a_hi_ref, xt_ref, w1_ref, b1_ref, w2_ref, b2_ref,
                out_ref, a8_ref, dinv_ref, m1t_ref, m2t_ref):
    i = pl.program_id(0)

    @pl.when(i < PH)
    def _():
        col = jax.lax.broadcasted_iota(jnp.int32, (BD, N), 1)
        row = jax.lax.broadcasted_iota(jnp.int32, (BD, N), 0) + 2 * i * BD
        a = a_lo_ref[...]
        ah_lo = jnp.where(row == col, 1.0, (a != 0).astype(jnp.float32))
        a8_ref[pl.ds(2 * i * BD, BD), :] = ah_lo.astype(jnp.bfloat16)
        a = a_hi_ref[...]
        ah_hi = jnp.where(row + BD == col, 1.0, (a != 0).astype(jnp.float32))
        a8_ref[pl.ds((2 * i + 1) * BD, BD), :] = ah_hi.astype(jnp.bfloat16)
        deg = (jnp.sum(ah_lo, axis=0, keepdims=True)
               + jnp.sum(ah_hi, axis=0, keepdims=True))
        @pl.when(i == 0)
        def _():
            dinv_ref[...] = deg
        @pl.when(i > 0)
        def _():
            dinv_ref[...] += deg

    @pl.when(i == PH - 1)
    def _():
        deg = dinv_ref[...]
        dinv_ref[...] = jnp.where(deg > 0, jax.lax.rsqrt(deg), 0.0)
        xwt = jax.lax.dot_general(
            w1_ref[...], xt_ref[...], (((0,), (0,)), ((), ())),
            preferred_element_type=jnp.float32)
        m1t_ref[...] = (dinv_ref[...] * xwt).astype(jnp.bfloat16)

    @pl.when(i == PH)
    def _():
        acc = jnp.dot(m1t_ref[...], a8_ref[...],
                      preferred_element_type=jnp.float32)
        dinv = dinv_ref[...]
        h = jnp.maximum(dinv * acc + b1_ref[...], 0.0)
        m2t = jax.lax.dot_general(
            w2_ref[...], h, (((0,), (0,)), ((), ())),
            preferred_element_type=jnp.float32)
        m2t_ref[...] = (dinv * m2t).astype(jnp.bfloat16)

    @pl.when(i == PH + 1)
    def _():
        acc = jnp.dot(m2t_ref[...], a8_ref[...],
                      preferred_element_type=jnp.float32)
        p = dinv_ref[...] * acc + b2_ref[...] + xt_ref[...]
        mx = jnp.max(p, axis=1, keepdims=True)
        e = jnp.exp(p - mx)
        s = jnp.sum(e, axis=1, keepdims=True)
        out_ref[...] = e / s


@jax.jit
def _run(A, xt, W1, b1c, W2, b2c):
    out_t = pl.pallas_call(
        _gcn_kernel,
        grid=(PH + 2,),
        in_specs=[
            pl.BlockSpec((BD, N),
                         lambda i: (jnp.minimum(2 * i, NBLK - 2), 0)),
            pl.BlockSpec((BD, N),
                         lambda i: (jnp.minimum(2 * i + 1, NBLK - 1), 0)),
            pl.BlockSpec((F, N), lambda i: (0, 0)),
            pl.BlockSpec((F, F), lambda i: (0, 0)),
            pl.BlockSpec((F, 1), lambda i: (0, 0)),
            pl.BlockSpec((F, F), lambda i: (0, 0)),
            pl.BlockSpec((F, 1), lambda i: (0, 0)),
        ],
        out_specs=pl.BlockSpec((F, N), lambda i: (0, 0)),
        out_shape=jax.ShapeDtypeStruct((F, N), jnp.float32),
        scratch_shapes=[
            pltpu.VMEM((N, N), jnp.bfloat16),
            pltpu.VMEM((1, N), jnp.float32),
            pltpu.VMEM((F, N), jnp.bfloat16),
            pltpu.VMEM((F, N), jnp.bfloat16),
        ],
    )(A, A, xt, W1, b1c, W2, b2c)
    return out_t


def kernel(A, x, W1, b1, W2, b2, sigmoid_param):
    out_t = _run(A, x.T, W1, b1.reshape(F, 1), W2, b2.reshape(F, 1))
    return out_t.T.astype(jnp.float64)
